# baseline (speedup 1.0000x reference)
import functools

import jax
import jax.numpy as jnp
from jax import lax
from jax.experimental import pallas as pl
from jax.experimental.pallas import tpu as pltpu
from jax.experimental.pallas import tpu_sc as plsc

NC = 2
NS = 16
NW = NC * NS
CHUNK = 128
LANES = 16


@functools.lru_cache(maxsize=None)
def _build_fuse(n, d, blk):
    def body(ne_ref, va_ref, ua_ref, t1_ref, t2_ref):
        x = ne_ref[...]
        t1_ref[...] = x + va_ref[...]
        t2_ref[...] = x + ua_ref[...]

    spec = pl.BlockSpec((blk, d), lambda i: (i, 0))
    return pl.pallas_call(
        body,
        grid=(n // blk,),
        in_specs=[spec, spec, spec],
        out_specs=[spec, spec],
        out_shape=[jax.ShapeDtypeStruct((n, d), jnp.float32)] * 2,
    )


@functools.lru_cache(maxsize=None)
def _build_sc_stage(n, d, p, bp):
    nchunk = bp // CHUNK
    npass = -(-nchunk // NW)
    pc = p * CHUNK
    ng = CHUNK // LANES

    def body(nodes_hbm, l1f_hbm, l2f_hbm, ne_hbm,
             self_hbm, c1_hbm, c2_hbm,
             nv, idx1, col1, col2, selfb, sem_i, sem_s):
        _c = lax.axis_index("c")
        _s = lax.axis_index("s")
        wid = lax.rem(_s + _c, 2) * NS + _s

        def pass_body(it, carry):
            c = it * NW + wid

            @pl.when(c < nchunk)
            def _():
                base = c * CHUNK
                pltpu.sync_copy(nodes_hbm.at[pl.ds(base, CHUNK)], nv)
                cps = pltpu.async_copy(ne_hbm.at[nv], selfb, sem_s)

                def build(j, c2):
                    def vec(g, c3):
                        v = nv[pl.ds(g * LANES, LANES)]
                        idx1[pl.ds(j * CHUNK + g * LANES, LANES)] = v + j * n
                        return c3

                    lax.fori_loop(0, ng, vec, 0)
                    sl = pl.ds(j * CHUNK, CHUNK)
                    pltpu.async_copy(l1f_hbm.at[idx1.at[sl]], col1.at[sl], sem_i)
                    pltpu.async_copy(l2f_hbm.at[idx1.at[sl]], col2.at[sl], sem_i)
                    return c2

                lax.fori_loop(0, p, build, 0)

                def drain_i(j, c2):
                    sl = pl.ds(0, CHUNK)
                    pltpu.make_async_copy(
                        l1f_hbm.at[idx1.at[sl]], col1.at[sl], sem_i).wait()
                    pltpu.make_async_copy(
                        l2f_hbm.at[idx1.at[sl]], col2.at[sl], sem_i).wait()
                    return c2

                lax.fori_loop(0, p, drain_i, 0)
                pltpu.sync_copy(col1, c1_hbm.at[pl.ds(c * pc, pc)])
                pltpu.sync_copy(col2, c2_hbm.at[pl.ds(c * pc, pc)])
                cps.wait()
                pltpu.sync_copy(selfb, self_hbm.at[pl.ds(base, CHUNK)])

            return carry

        lax.fori_loop(0, npass, pass_body, 0)

    mesh = plsc.VectorSubcoreMesh(
        core_axis_name="c", subcore_axis_name="s", num_cores=NC, num_subcores=NS)
    return pl.kernel(
        body,
        out_type=(jax.ShapeDtypeStruct((bp, d), jnp.float32),
                  jax.ShapeDtypeStruct((nchunk * pc,), jnp.int32),
                  jax.ShapeDtypeStruct((nchunk * pc,), jnp.int32)),
        mesh=mesh,
        scratch_types=[
            pltpu.VMEM((CHUNK,), jnp.int32),
            pltpu.VMEM((pc,), jnp.int32),
            pltpu.VMEM((pc,), jnp.int32),
            pltpu.VMEM((pc,), jnp.int32),
            pltpu.VMEM((CHUNK, d), jnp.float32),
            pltpu.SemaphoreType.DMA,
            pltpu.SemaphoreType.DMA,
        ],
    )


@functools.lru_cache(maxsize=None)
def _build_sc_add(n, d, p, bp):
    nchunk = bp // CHUNK
    npass = -(-nchunk // NW)
    pc = p * CHUNK

    n_slow = (nchunk * 14) // 39

    def body(c1_hbm, c2_hbm, t1_hbm, t2_hbm,
             s1_hbm, s2_hbm,
             col1, col2, acc1, acc2, sem_i, sem_a, sem_b):
        _c = lax.axis_index("c")
        _s = lax.axis_index("s")
        n_fast = nchunk - n_slow
        lo = jnp.where(_c == 0, (_s * n_slow) // NS,
                       n_slow + (_s * n_fast) // NS)
        hi = jnp.where(_c == 0, ((_s + 1) * n_slow) // NS,
                       n_slow + ((_s + 1) * n_fast) // NS)

        def pass_body(c, carry):
            if True:
                base = c * CHUNK
                cp1 = pltpu.async_copy(c1_hbm.at[pl.ds(c * pc, pc)], col1, sem_i)
                cp2 = pltpu.async_copy(c2_hbm.at[pl.ds(c * pc, pc)], col2, sem_i)
                cp1.wait()
                cp2.wait()

                pltpu.async_copy(
                    t1_hbm.at[col1.at[pl.ds(0, CHUNK)]], acc1, sem_a).wait()
                pltpu.async_copy(
                    t2_hbm.at[col2.at[pl.ds(0, CHUNK)]], acc2, sem_b).wait()

                def fire(j, c2):
                    sl = pl.ds(j * CHUNK, CHUNK)
                    pltpu.async_copy(t1_hbm.at[col1.at[sl]], acc1, sem_a,
                                     add=True)
                    pltpu.async_copy(t2_hbm.at[col2.at[sl]], acc2, sem_b,
                                     add=True)
                    return c2

                lax.fori_loop(1, p, fire, 0)

                def drain_a(j, c2):
                    sl = pl.ds(0, CHUNK)
                    pltpu.make_async_copy(
                        t1_hbm.at[col1.at[sl]], acc1, sem_a).wait()
                    pltpu.make_async_copy(
                        t2_hbm.at[col2.at[sl]], acc2, sem_b).wait()
                    return c2

                lax.fori_loop(1, p, drain_a, 0)

                pltpu.sync_copy(acc1, s1_hbm.at[pl.ds(base, CHUNK)])
                pltpu.sync_copy(acc2, s2_hbm.at[pl.ds(base, CHUNK)])

            return carry

        lax.fori_loop(lo, hi, pass_body, 0)

    mesh = plsc.VectorSubcoreMesh(
        core_axis_name="c", subcore_axis_name="s", num_cores=NC, num_subcores=NS)
    return pl.kernel(
        body,
        out_type=(jax.ShapeDtypeStruct((bp, d), jnp.float32),
                  jax.ShapeDtypeStruct((bp, d), jnp.float32)),
        mesh=mesh,
        scratch_types=[
            pltpu.VMEM((pc,), jnp.int32),
            pltpu.VMEM((pc,), jnp.int32),
            pltpu.VMEM((CHUNK, d), jnp.float32),
            pltpu.VMEM((CHUNK, d), jnp.float32),
            pltpu.SemaphoreType.DMA,
            pltpu.SemaphoreType.DMA,
            pltpu.SemaphoreType.DMA,
        ],
    )


@functools.lru_cache(maxsize=None)
def _build_combine(b, bp, d, p, blk):
    inv_p = 1.0 / p

    def body(sf_ref, s1_ref, s2_ref, aw_ref, bw_ref, gt_ref, gb_ref,
             ab_ref, bb_ref, out_ref):
        sf = sf_ref[...]
        l1 = s1_ref[...] * inv_p
        l2 = s2_ref[...] * inv_p
        aw = aw_ref[...]
        bw = bw_ref[...]
        alpha = (jnp.sum(sf * aw[:, :d], axis=1, keepdims=True)
                 + jnp.sum(l1 * aw[:, d:], axis=1, keepdims=True) + ab_ref[0])
        beta = (jnp.sum(sf * bw[:, :d], axis=1, keepdims=True)
                + jnp.sum(l2 * bw[:, d:], axis=1, keepdims=True) + bb_ref[0])
        comb = sf + alpha * l1 + beta * l2
        out_ref[...] = (jnp.dot(comb, gt_ref[...],
                                preferred_element_type=jnp.float32)
                        + gb_ref[...])

    row = pl.BlockSpec((blk, d), lambda i: (i, 0))
    return pl.pallas_call(
        body,
        grid=(b // blk,),
        in_specs=[
            row, row, row,
            pl.BlockSpec((1, 2 * d), lambda i: (0, 0)),
            pl.BlockSpec((1, 2 * d), lambda i: (0, 0)),
            pl.BlockSpec((d, d), lambda i: (0, 0)),
            pl.BlockSpec((1, d), lambda i: (0, 0)),
            pl.BlockSpec(memory_space=pltpu.SMEM),
            pl.BlockSpec(memory_space=pltpu.SMEM),
        ],
        out_specs=row,
        out_shape=jax.ShapeDtypeStruct((b, d), jnp.float32),
    )


def kernel(nodes, node2e_weight, l1paths, l2paths, va, ua,
           alpha_w, alpha_b, beta_w, beta_b, gamma_w, gamma_b):
    n, d = node2e_weight.shape
    b = nodes.shape[0]
    p = l1paths.shape[1]
    bp = -(-b // CHUNK) * CHUNK

    t1, t2 = _build_fuse(n, d, 2000)(node2e_weight, va, ua)

    nodes_p = jnp.zeros((bp,), jnp.int32).at[:b].set(nodes.astype(jnp.int32))
    l1f = l1paths[:, :, 1].astype(jnp.int32).T.reshape(-1)
    l2f = l2paths[:, :, 2].astype(jnp.int32).T.reshape(-1)

    selfb, c1, c2 = _build_sc_stage(n, d, p, bp)(
        nodes_p, l1f, l2f, node2e_weight)
    s1, s2 = _build_sc_add(n, d, p, bp)(c1, c2, t1, t2)

    return _build_combine(b, bp, d, p, 2000)(
        selfb, s1, s2, alpha_w, beta_w, gamma_w.T, gamma_b.reshape(1, d),
        alpha_b, beta_b)


# speedup vs baseline: 1.0912x; 1.0912x over previous
import functools

import jax
import jax.numpy as jnp
from jax import lax
from jax.experimental import pallas as pl
from jax.experimental.pallas import tpu as pltpu
from jax.experimental.pallas import tpu_sc as plsc

NC = 2
NS = 16
NW = NC * NS
CHUNK = 128
LANES = 16


@functools.lru_cache(maxsize=None)
def _build_fuse(n, d, blk):
    def body(ne_ref, va_ref, ua_ref, t1_ref, t2_ref):
        x = ne_ref[...]
        t1_ref[...] = x + va_ref[...]
        t2_ref[...] = x + ua_ref[...]

    spec = pl.BlockSpec((blk, d), lambda i: (i, 0))
    return pl.pallas_call(
        body,
        grid=(n // blk,),
        in_specs=[spec, spec, spec],
        out_specs=[spec, spec],
        out_shape=[jax.ShapeDtypeStruct((n, d), jnp.float32)] * 2,
    )


@functools.lru_cache(maxsize=None)
def _build_sc_stage(n, d, p, bp):
    nchunk = bp // CHUNK
    npass = -(-nchunk // NW)
    pc = p * CHUNK
    ng = CHUNK // LANES

    def body(nodes_hbm, l1f_hbm, l2f_hbm, ne_hbm,
             self_hbm, c1_hbm, c2_hbm,
             nv, idx1, col1, col2, selfb, sem_i, sem_s):
        _c = lax.axis_index("c")
        _s = lax.axis_index("s")
        wid = lax.rem(_s + _c, 2) * NS + _s

        def pass_body(it, carry):
            c = it * NW + wid

            @pl.when(c < nchunk)
            def _():
                base = c * CHUNK
                pltpu.sync_copy(nodes_hbm.at[pl.ds(base, CHUNK)], nv)
                cps = pltpu.async_copy(ne_hbm.at[nv], selfb, sem_s)

                def build(j, c2):
                    def vec(g, c3):
                        v = nv[pl.ds(g * LANES, LANES)]
                        idx1[pl.ds(j * CHUNK + g * LANES, LANES)] = v + j * n
                        return c3

                    lax.fori_loop(0, ng, vec, 0)
                    sl = pl.ds(j * CHUNK, CHUNK)
                    pltpu.async_copy(l1f_hbm.at[idx1.at[sl]], col1.at[sl], sem_i)
                    pltpu.async_copy(l2f_hbm.at[idx1.at[sl]], col2.at[sl], sem_i)
                    return c2

                lax.fori_loop(0, p, build, 0)

                def drain_i(j, c2):
                    sl = pl.ds(0, CHUNK)
                    pltpu.make_async_copy(
                        l1f_hbm.at[idx1.at[sl]], col1.at[sl], sem_i).wait()
                    pltpu.make_async_copy(
                        l2f_hbm.at[idx1.at[sl]], col2.at[sl], sem_i).wait()
                    return c2

                lax.fori_loop(0, p, drain_i, 0)
                pltpu.sync_copy(col1, c1_hbm.at[pl.ds(c * pc, pc)])
                pltpu.sync_copy(col2, c2_hbm.at[pl.ds(c * pc, pc)])
                cps.wait()
                pltpu.sync_copy(selfb, self_hbm.at[pl.ds(base, CHUNK)])

            return carry

        lax.fori_loop(0, npass, pass_body, 0)

    mesh = plsc.VectorSubcoreMesh(
        core_axis_name="c", subcore_axis_name="s", num_cores=NC, num_subcores=NS)
    return pl.kernel(
        body,
        out_type=(jax.ShapeDtypeStruct((bp, d), jnp.float32),
                  jax.ShapeDtypeStruct((nchunk * pc,), jnp.int32),
                  jax.ShapeDtypeStruct((nchunk * pc,), jnp.int32)),
        mesh=mesh,
        scratch_types=[
            pltpu.VMEM((CHUNK,), jnp.int32),
            pltpu.VMEM((pc,), jnp.int32),
            pltpu.VMEM((pc,), jnp.int32),
            pltpu.VMEM((pc,), jnp.int32),
            pltpu.VMEM((CHUNK, d), jnp.float32),
            pltpu.SemaphoreType.DMA,
            pltpu.SemaphoreType.DMA,
        ],
    )


@functools.lru_cache(maxsize=None)
def _build_sc_add(n, d, p, bp):
    nchunk = bp // CHUNK
    npass = -(-nchunk // NW)
    pc = p * CHUNK

    def body(c1_hbm, c2_hbm, t1_hbm, t2_hbm,
             s1_hbm, s2_hbm,
             col1, col2, acc1, acc2, sem_i, sem_a, sem_b):
        _c = lax.axis_index("c")
        _s = lax.axis_index("s")
        wid = lax.rem(_s + _c, 2) * NS + _s

        def pass_body(it, carry):
            c = it * NW + wid

            @pl.when(c < nchunk)
            def _():
                base = c * CHUNK
                cp1 = pltpu.async_copy(c1_hbm.at[pl.ds(c * pc, pc)], col1, sem_i)
                cp2 = pltpu.async_copy(c2_hbm.at[pl.ds(c * pc, pc)], col2, sem_i)
                cp1.wait()
                cp2.wait()

                j0a = pltpu.async_copy(
                    t1_hbm.at[col1.at[pl.ds(0, CHUNK)]], acc1, sem_a)
                j0b = pltpu.async_copy(
                    t2_hbm.at[col2.at[pl.ds(0, CHUNK)]], acc2, sem_b)
                j0a.wait()
                j0b.wait()

                def fire(j, c2):
                    sl = pl.ds(j * CHUNK, CHUNK)
                    pltpu.async_copy(t1_hbm.at[col1.at[sl]], acc1, sem_a,
                                     add=True)
                    pltpu.async_copy(t2_hbm.at[col2.at[sl]], acc2, sem_b,
                                     add=True)
                    return c2

                lax.fori_loop(1, p, fire, 0)

                def drain_a(j, c2):
                    sl = pl.ds(0, CHUNK)
                    pltpu.make_async_copy(
                        t1_hbm.at[col1.at[sl]], acc1, sem_a).wait()
                    pltpu.make_async_copy(
                        t2_hbm.at[col2.at[sl]], acc2, sem_b).wait()
                    return c2

                lax.fori_loop(1, p, drain_a, 0)

                pltpu.sync_copy(acc1, s1_hbm.at[pl.ds(base, CHUNK)])
                pltpu.sync_copy(acc2, s2_hbm.at[pl.ds(base, CHUNK)])

            return carry

        lax.fori_loop(0, npass, pass_body, 0)

    mesh = plsc.VectorSubcoreMesh(
        core_axis_name="c", subcore_axis_name="s", num_cores=NC, num_subcores=NS)
    return pl.kernel(
        body,
        out_type=(jax.ShapeDtypeStruct((bp, d), jnp.float32),
                  jax.ShapeDtypeStruct((bp, d), jnp.float32)),
        mesh=mesh,
        scratch_types=[
            pltpu.VMEM((pc,), jnp.int32),
            pltpu.VMEM((pc,), jnp.int32),
            pltpu.VMEM((CHUNK, d), jnp.float32),
            pltpu.VMEM((CHUNK, d), jnp.float32),
            pltpu.SemaphoreType.DMA,
            pltpu.SemaphoreType.DMA,
            pltpu.SemaphoreType.DMA,
        ],
    )


@functools.lru_cache(maxsize=None)
def _build_combine(b, bp, d, p, blk):
    inv_p = 1.0 / p

    def body(sf_ref, s1_ref, s2_ref, aw_ref, bw_ref, gt_ref, gb_ref,
             ab_ref, bb_ref, out_ref):
        sf = sf_ref[...]
        l1 = s1_ref[...] * inv_p
        l2 = s2_ref[...] * inv_p
        aw = aw_ref[...]
        bw = bw_ref[...]
        alpha = (jnp.sum(sf * aw[:, :d], axis=1, keepdims=True)
                 + jnp.sum(l1 * aw[:, d:], axis=1, keepdims=True) + ab_ref[0])
        beta = (jnp.sum(sf * bw[:, :d], axis=1, keepdims=True)
                + jnp.sum(l2 * bw[:, d:], axis=1, keepdims=True) + bb_ref[0])
        comb = sf + alpha * l1 + beta * l2
        out_ref[...] = (jnp.dot(comb, gt_ref[...],
                                preferred_element_type=jnp.float32)
                        + gb_ref[...])

    row = pl.BlockSpec((blk, d), lambda i: (i, 0))
    return pl.pallas_call(
        body,
        grid=(b // blk,),
        in_specs=[
            row, row, row,
            pl.BlockSpec((1, 2 * d), lambda i: (0, 0)),
            pl.BlockSpec((1, 2 * d), lambda i: (0, 0)),
            pl.BlockSpec((d, d), lambda i: (0, 0)),
            pl.BlockSpec((1, d), lambda i: (0, 0)),
            pl.BlockSpec(memory_space=pltpu.SMEM),
            pl.BlockSpec(memory_space=pltpu.SMEM),
        ],
        out_specs=row,
        out_shape=jax.ShapeDtypeStruct((b, d), jnp.float32),
    )


def kernel(nodes, node2e_weight, l1paths, l2paths, va, ua,
           alpha_w, alpha_b, beta_w, beta_b, gamma_w, gamma_b):
    n, d = node2e_weight.shape
    b = nodes.shape[0]
    p = l1paths.shape[1]
    bp = -(-b // CHUNK) * CHUNK

    t1, t2 = _build_fuse(n, d, 2000)(node2e_weight, va, ua)

    nodes_p = jnp.zeros((bp,), jnp.int32).at[:b].set(nodes.astype(jnp.int32))
    l1f = l1paths[:, :, 1].astype(jnp.int32).T.reshape(-1)
    l2f = l2paths[:, :, 2].astype(jnp.int32).T.reshape(-1)

    selfb, c1, c2 = _build_sc_stage(n, d, p, bp)(
        nodes_p, l1f, l2f, node2e_weight)
    s1, s2 = _build_sc_add(n, d, p, bp)(c1, c2, t1, t2)

    return _build_combine(b, bp, d, p, 2000)(
        selfb, s1, s2, alpha_w, beta_w, gamma_w.T, gamma_b.reshape(1, d),
        alpha_b, beta_b)


# speedup vs baseline: 1.0963x; 1.0047x over previous
import functools

import jax
import jax.numpy as jnp
from jax import lax
from jax.experimental import pallas as pl
from jax.experimental.pallas import tpu as pltpu
from jax.experimental.pallas import tpu_sc as plsc

NC = 2
NS = 16
NW = NC * NS
CHUNK = 128
LANES = 16


@functools.lru_cache(maxsize=None)
def _build_fuse(n, d, blk):
    def body(ne_ref, va_ref, ua_ref, t1_ref, t2_ref):
        x = ne_ref[...]
        t1_ref[...] = x + va_ref[...]
        t2_ref[...] = x + ua_ref[...]

    spec = pl.BlockSpec((blk, d), lambda i: (i, 0))
    return pl.pallas_call(
        body,
        grid=(n // blk,),
        in_specs=[spec, spec, spec],
        out_specs=[spec, spec],
        out_shape=[jax.ShapeDtypeStruct((n, d), jnp.float32)] * 2,
    )


@functools.lru_cache(maxsize=None)
def _build_sc_stage(n, d, p, bp):
    nchunk = bp // CHUNK
    npass = -(-nchunk // NW)
    pc = p * CHUNK
    ng = CHUNK // LANES

    def body(nodes_hbm, l1f_hbm, l2f_hbm, ne_hbm,
             self_hbm, c1_hbm, c2_hbm,
             nv, idx1, col1, col2, selfb, sem_i, sem_s):
        _c = lax.axis_index("c")
        _s = lax.axis_index("s")
        wid = lax.rem(_s + _c, 2) * NS + _s

        def pass_body(it, carry):
            c = it * NW + wid

            @pl.when(c < nchunk)
            def _():
                base = c * CHUNK
                pltpu.sync_copy(nodes_hbm.at[pl.ds(base, CHUNK)], nv)
                cps = pltpu.async_copy(ne_hbm.at[nv], selfb, sem_s)

                def build(j, c2):
                    def vec(g, c3):
                        v = nv[pl.ds(g * LANES, LANES)]
                        idx1[pl.ds(j * CHUNK + g * LANES, LANES)] = v + j * n
                        return c3

                    lax.fori_loop(0, ng, vec, 0)
                    sl = pl.ds(j * CHUNK, CHUNK)
                    pltpu.async_copy(l1f_hbm.at[idx1.at[sl]], col1.at[sl], sem_i)
                    pltpu.async_copy(l2f_hbm.at[idx1.at[sl]], col2.at[sl], sem_i)
                    return c2

                lax.fori_loop(0, p, build, 0)

                def drain_i(j, c2):
                    sl = pl.ds(0, CHUNK)
                    pltpu.make_async_copy(
                        l1f_hbm.at[idx1.at[sl]], col1.at[sl], sem_i).wait()
                    pltpu.make_async_copy(
                        l2f_hbm.at[idx1.at[sl]], col2.at[sl], sem_i).wait()
                    return c2

                lax.fori_loop(0, p, drain_i, 0)
                pltpu.sync_copy(col1, c1_hbm.at[pl.ds(c * pc, pc)])
                pltpu.sync_copy(col2, c2_hbm.at[pl.ds(c * pc, pc)])
                cps.wait()
                pltpu.sync_copy(selfb, self_hbm.at[pl.ds(base, CHUNK)])

            return carry

        lax.fori_loop(0, npass, pass_body, 0)

    mesh = plsc.VectorSubcoreMesh(
        core_axis_name="c", subcore_axis_name="s", num_cores=NC, num_subcores=NS)
    return pl.kernel(
        body,
        out_type=(jax.ShapeDtypeStruct((bp, d), jnp.float32),
                  jax.ShapeDtypeStruct((nchunk * pc,), jnp.int32),
                  jax.ShapeDtypeStruct((nchunk * pc,), jnp.int32)),
        mesh=mesh,
        scratch_types=[
            pltpu.VMEM((CHUNK,), jnp.int32),
            pltpu.VMEM((pc,), jnp.int32),
            pltpu.VMEM((pc,), jnp.int32),
            pltpu.VMEM((pc,), jnp.int32),
            pltpu.VMEM((CHUNK, d), jnp.float32),
            pltpu.SemaphoreType.DMA,
            pltpu.SemaphoreType.DMA,
        ],
    )


@functools.lru_cache(maxsize=None)
def _build_sc_add(n, d, p, bp):
    nchunk = bp // CHUNK
    npass = -(-nchunk // NW)
    pc = p * CHUNK

    def body(c1_hbm, c2_hbm, t1_hbm, t2_hbm,
             s1_hbm, s2_hbm,
             col1, col2, acc1, acc2, sem_i, sem_a, sem_b):
        _c = lax.axis_index("c")
        _s = lax.axis_index("s")
        wid = lax.rem(_s + _c, 2) * NS + _s

        @pl.when(wid < nchunk)
        def _():
            pltpu.async_copy(c1_hbm.at[pl.ds(wid * pc, pc)],
                             col1.at[pl.ds(0, pc)], sem_i)
            pltpu.async_copy(c2_hbm.at[pl.ds(wid * pc, pc)],
                             col2.at[pl.ds(0, pc)], sem_i)

        def pass_body(it, carry):
            c = it * NW + wid
            off = lax.rem(it, 2) * pc

            @pl.when(c < nchunk)
            def _():
                base = c * CHUNK
                sl0 = pl.ds(0, pc)
                pltpu.make_async_copy(
                    c1_hbm.at[sl0], col1.at[sl0], sem_i).wait()
                pltpu.make_async_copy(
                    c2_hbm.at[sl0], col2.at[sl0], sem_i).wait()

                j0a = pltpu.async_copy(
                    t1_hbm.at[col1.at[pl.ds(off, CHUNK)]], acc1, sem_a)
                j0b = pltpu.async_copy(
                    t2_hbm.at[col2.at[pl.ds(off, CHUNK)]], acc2, sem_b)
                j0a.wait()
                j0b.wait()

                def fire(j, c2):
                    sl = pl.ds(off + j * CHUNK, CHUNK)
                    pltpu.async_copy(t1_hbm.at[col1.at[sl]], acc1, sem_a,
                                     add=True)
                    pltpu.async_copy(t2_hbm.at[col2.at[sl]], acc2, sem_b,
                                     add=True)
                    return c2

                lax.fori_loop(1, p, fire, 0)

                nc_ = (it + 1) * NW + wid

                @pl.when(nc_ < nchunk)
                def _():
                    noff = pc - off
                    pltpu.async_copy(c1_hbm.at[pl.ds(nc_ * pc, pc)],
                                     col1.at[pl.ds(noff, pc)], sem_i)
                    pltpu.async_copy(c2_hbm.at[pl.ds(nc_ * pc, pc)],
                                     col2.at[pl.ds(noff, pc)], sem_i)

                def drain_a(j, c2):
                    sl = pl.ds(0, CHUNK)
                    pltpu.make_async_copy(
                        t1_hbm.at[col1.at[sl]], acc1, sem_a).wait()
                    pltpu.make_async_copy(
                        t2_hbm.at[col2.at[sl]], acc2, sem_b).wait()
                    return c2

                lax.fori_loop(1, p, drain_a, 0)

                pltpu.sync_copy(acc1, s1_hbm.at[pl.ds(base, CHUNK)])
                pltpu.sync_copy(acc2, s2_hbm.at[pl.ds(base, CHUNK)])

            return carry

        lax.fori_loop(0, npass, pass_body, 0)

    mesh = plsc.VectorSubcoreMesh(
        core_axis_name="c", subcore_axis_name="s", num_cores=NC, num_subcores=NS)
    return pl.kernel(
        body,
        out_type=(jax.ShapeDtypeStruct((bp, d), jnp.float32),
                  jax.ShapeDtypeStruct((bp, d), jnp.float32)),
        mesh=mesh,
        scratch_types=[
            pltpu.VMEM((2 * pc,), jnp.int32),
            pltpu.VMEM((2 * pc,), jnp.int32),
            pltpu.VMEM((CHUNK, d), jnp.float32),
            pltpu.VMEM((CHUNK, d), jnp.float32),
            pltpu.SemaphoreType.DMA,
            pltpu.SemaphoreType.DMA,
            pltpu.SemaphoreType.DMA,
        ],
    )


@functools.lru_cache(maxsize=None)
def _build_combine(b, bp, d, p, blk):
    inv_p = 1.0 / p

    def body(sf_ref, s1_ref, s2_ref, aw_ref, bw_ref, gt_ref, gb_ref,
             ab_ref, bb_ref, out_ref):
        sf = sf_ref[...]
        l1 = s1_ref[...] * inv_p
        l2 = s2_ref[...] * inv_p
        aw = aw_ref[...]
        bw = bw_ref[...]
        alpha = (jnp.sum(sf * aw[:, :d], axis=1, keepdims=True)
                 + jnp.sum(l1 * aw[:, d:], axis=1, keepdims=True) + ab_ref[0])
        beta = (jnp.sum(sf * bw[:, :d], axis=1, keepdims=True)
                + jnp.sum(l2 * bw[:, d:], axis=1, keepdims=True) + bb_ref[0])
        comb = sf + alpha * l1 + beta * l2
        out_ref[...] = (jnp.dot(comb, gt_ref[...],
                                preferred_element_type=jnp.float32)
                        + gb_ref[...])

    row = pl.BlockSpec((blk, d), lambda i: (i, 0))
    return pl.pallas_call(
        body,
        grid=(b // blk,),
        in_specs=[
            row, row, row,
            pl.BlockSpec((1, 2 * d), lambda i: (0, 0)),
            pl.BlockSpec((1, 2 * d), lambda i: (0, 0)),
            pl.BlockSpec((d, d), lambda i: (0, 0)),
            pl.BlockSpec((1, d), lambda i: (0, 0)),
            pl.BlockSpec(memory_space=pltpu.SMEM),
            pl.BlockSpec(memory_space=pltpu.SMEM),
        ],
        out_specs=row,
        out_shape=jax.ShapeDtypeStruct((b, d), jnp.float32),
    )


def kernel(nodes, node2e_weight, l1paths, l2paths, va, ua,
           alpha_w, alpha_b, beta_w, beta_b, gamma_w, gamma_b):
    n, d = node2e_weight.shape
    b = nodes.shape[0]
    p = l1paths.shape[1]
    bp = -(-b // CHUNK) * CHUNK

    t1, t2 = _build_fuse(n, d, 2000)(node2e_weight, va, ua)

    nodes_p = jnp.zeros((bp,), jnp.int32).at[:b].set(nodes.astype(jnp.int32))
    l1f = l1paths[:, :, 1].astype(jnp.int32).T.reshape(-1)
    l2f = l2paths[:, :, 2].astype(jnp.int32).T.reshape(-1)

    selfb, c1, c2 = _build_sc_stage(n, d, p, bp)(
        nodes_p, l1f, l2f, node2e_weight)
    s1, s2 = _build_sc_add(n, d, p, bp)(c1, c2, t1, t2)

    return _build_combine(b, bp, d, p, 2000)(
        selfb, s1, s2, alpha_w, beta_w, gamma_w.T, gamma_b.reshape(1, d),
        alpha_b, beta_b)
